# baseline (speedup 1.0000x reference)
import functools

import jax
import jax.numpy as jnp
from jax import lax
from jax.experimental import pallas as pl
from jax.experimental.pallas import tpu as pltpu
from jax.experimental.pallas import tpu_sc as plsc

B, L, H = 1024, 200, 64
N = B * L
EPS = 1e-05

NC, NS, LANES = 2, 16, 16
NW = NC * NS
CHUNK = 128
GROUPS = CHUNK // LANES
PER_W = N // NW
NCH = PER_W // CHUNK
MAXLEN, TYPE_VOCAB = 200, 2


def _rsqrt(v):
    i = lax.bitcast_convert_type(v, jnp.int32)
    i = jnp.int32(0x5F3759DF) - lax.shift_right_logical(i, 1)
    y = lax.bitcast_convert_type(i, jnp.float32)
    half = v * 0.5
    for _ in range(3):
        y = y * (1.5 - half * y * y)
    return y


_mesh = plsc.VectorSubcoreMesh(core_axis_name="c", subcore_axis_name="s")


@functools.partial(
    pl.kernel,
    mesh=_mesh,
    out_type=jax.ShapeDtypeStruct((N, H), jnp.float32),
    compiler_params=pltpu.CompilerParams(
        needs_layout_passes=False, use_tc_tiling_on_sc=False),
    scratch_types=[
        pltpu.VMEM((PER_W,), jnp.int32),
        pltpu.VMEM((PER_W,), jnp.int32),
        pltpu.VMEM((PER_W,), jnp.int32),
        pltpu.VMEM((CHUNK, H), jnp.float32),
        pltpu.VMEM((MAXLEN, H), jnp.float32),
        pltpu.VMEM((TYPE_VOCAB, H), jnp.float32),
        pltpu.VMEM((H, LANES), jnp.float32),
        pltpu.VMEM((H,), jnp.float32),
        pltpu.VMEM((H,), jnp.float32),
        pltpu.SemaphoreType.DMA,
    ],
)
def _sc_embed(x_hbm, pos_hbm, sent_hbm, word_hbm, posw_hbm, sentw_hbm,
              gamma_hbm, beta_hbm, out_hbm,
              idx_w, idx_p, idx_s, rows_w, posw_v, sentw_v, tr_buf,
              g_v, b_v, sem):
    wid = lax.axis_index("s") * NC + lax.axis_index("c")

    pltpu.sync_copy(x_hbm.at[pl.ds(wid * PER_W, PER_W)], idx_w)
    pltpu.sync_copy(pos_hbm.at[pl.ds(wid * PER_W, PER_W)], idx_p)
    pltpu.sync_copy(sent_hbm.at[pl.ds(wid * PER_W, PER_W)], idx_s)
    pltpu.sync_copy(posw_hbm, posw_v)
    pltpu.sync_copy(sentw_hbm, sentw_v)
    pltpu.sync_copy(gamma_hbm, g_v)
    pltpu.sync_copy(beta_hbm, b_v)

    lane_iota = lax.iota(jnp.int32, LANES)
    g_regs = [g_v[pl.ds(j * LANES, LANES)] for j in range(H // LANES)]
    b_regs = [b_v[pl.ds(j * LANES, LANES)] for j in range(H // LANES)]

    def chunk_body(c, carry):
        isl = pl.ds(c * CHUNK, CHUNK)
        pltpu.async_copy(word_hbm.at[idx_w.at[isl]], rows_w, sem).wait()

        def group_body(g, gc):
            t0 = c * CHUNK + g * LANES
            rowv = g * LANES + lane_iota
            pv = idx_p[pl.ds(t0, LANES)]
            sv = idx_s[pl.ds(t0, LANES)]
            ssum = jnp.zeros((LANES,), jnp.float32)
            ssq = jnp.zeros((LANES,), jnp.float32)
            for h in range(H):
                hv = jnp.full((LANES,), h, jnp.int32)
                w = plsc.load_gather(rows_w, [rowv, hv])
                p = plsc.load_gather(posw_v, [pv, hv])
                s = plsc.load_gather(sentw_v, [sv, hv])
                acc = (w + p) + s
                tr_buf[h] = acc
                ssum = ssum + acc
                ssq = ssq + acc * acc
            mean = ssum * (1.0 / H)
            var = ssq * (1.0 / H) - mean * mean
            inv = _rsqrt(var + EPS)
            for h in range(H):
                hv = jnp.full((LANES,), h, jnp.int32)
                gs = g_regs[h // LANES][h % LANES]
                bs = b_regs[h // LANES][h % LANES]
                o = (tr_buf[h] - mean) * inv * gs + bs
                plsc.store_scatter(rows_w, [rowv, hv], o)
            return gc

        lax.fori_loop(0, GROUPS, group_body, 0)
        pltpu.sync_copy(rows_w,
                        out_hbm.at[pl.ds((wid * NCH + c) * CHUNK, CHUNK)])
        return carry

    lax.fori_loop(0, NCH, chunk_body, 0)


def kernel(x, pos_ids, sent_ids, word_W, pos_W, sent_W, gamma, beta):
    x2 = x.reshape(N).astype(jnp.int32)
    p2 = pos_ids.reshape(N).astype(jnp.int32)
    s2 = sent_ids.reshape(N).astype(jnp.int32)
    out = _sc_embed(x2, p2, s2, word_W, pos_W, sent_W, gamma, beta)
    return out.reshape(B, L, H)


# speedup vs baseline: 1.2076x; 1.2076x over previous
import functools

import jax
import jax.numpy as jnp
from jax import lax
from jax.experimental import pallas as pl
from jax.experimental.pallas import tpu as pltpu
from jax.experimental.pallas import tpu_sc as plsc

B, L, H = 1024, 200, 64
N = B * L
EPS = 1e-05

NC, NS, LANES = 2, 16, 16
NW = NC * NS
CHUNK = 128
GROUPS = CHUNK // LANES
PER_W = N // NW
NCH = PER_W // CHUNK
HREG = H // LANES
MAXLEN, TYPE_VOCAB = 200, 2
NPS = MAXLEN * TYPE_VOCAB


def _rsqrt(v):
    i = lax.bitcast_convert_type(v, jnp.int32)
    i = jnp.int32(0x5F3759DF) - lax.shift_right_logical(i, 1)
    y = lax.bitcast_convert_type(i, jnp.float32)
    half = v * 0.5
    for _ in range(3):
        y = y * (1.5 - half * y * y)
    return y


_mesh = plsc.VectorSubcoreMesh(core_axis_name="c", subcore_axis_name="s")


@functools.partial(
    pl.kernel,
    mesh=_mesh,
    out_type=jax.ShapeDtypeStruct((N, H), jnp.float32),
    compiler_params=pltpu.CompilerParams(
        needs_layout_passes=False, use_tc_tiling_on_sc=False),
    scratch_types=[
        pltpu.VMEM((PER_W,), jnp.int32),
        pltpu.VMEM((PER_W,), jnp.int32),
        pltpu.VMEM((PER_W,), jnp.int32),
        pltpu.VMEM((CHUNK, H), jnp.float32),
        pltpu.VMEM((CHUNK, H), jnp.float32),
        pltpu.VMEM((CHUNK, H), jnp.float32),
        pltpu.VMEM((CHUNK, H), jnp.float32),
        pltpu.VMEM((MAXLEN, H), jnp.float32),
        pltpu.VMEM((TYPE_VOCAB, H), jnp.float32),
        pltpu.VMEM((NPS * H,), jnp.float32),
        pltpu.VMEM((H * LANES,), jnp.float32),
        pltpu.VMEM((H,), jnp.float32),
        pltpu.VMEM((H,), jnp.float32),
        pltpu.SemaphoreType.DMA,
        pltpu.SemaphoreType.DMA,
        pltpu.SemaphoreType.DMA,
        pltpu.SemaphoreType.DMA,
    ],
)
def _sc_embed(x_hbm, pos_hbm, sent_hbm, word_hbm, posw_hbm, sentw_hbm,
              gamma_hbm, beta_hbm, out_hbm,
              idx_w, idx_p, idx_s, rows0, rows1, obuf0, obuf1,
              posw_v, sentw_v, ps_v, tr_v, g_v, b_v,
              gsem0, gsem1, osem0, osem1):
    wid = lax.axis_index("s") * NC + lax.axis_index("c")
    rows = (rows0, rows1)
    obuf = (obuf0, obuf1)
    gsem = (gsem0, gsem1)
    osem = (osem0, osem1)

    pltpu.sync_copy(x_hbm.at[pl.ds(wid * PER_W, PER_W)], idx_w)
    pltpu.sync_copy(pos_hbm.at[pl.ds(wid * PER_W, PER_W)], idx_p)
    pltpu.sync_copy(sent_hbm.at[pl.ds(wid * PER_W, PER_W)], idx_s)
    pltpu.sync_copy(posw_hbm, posw_v)
    pltpu.sync_copy(sentw_hbm, sentw_v)
    pltpu.sync_copy(gamma_hbm, g_v)
    pltpu.sync_copy(beta_hbm, b_v)

    lane_iota = lax.iota(jnp.int32, LANES)
    g_regs = [g_v[pl.ds(j * LANES, LANES)] for j in range(HREG)]
    b_regs = [b_v[pl.ds(j * LANES, LANES)] for j in range(HREG)]

    def ps_body(p, carry):
        for s in range(TYPE_VOCAB):
            base = (p * TYPE_VOCAB + s) * H
            for j in range(HREG):
                sl = pl.ds(j * LANES, LANES)
                ps_v[pl.ds(base + j * LANES, LANES)] = \
                    posw_v[p, sl] + sentw_v[s, sl]
        return carry

    lax.fori_loop(0, MAXLEN, ps_body, 0)

    def issue_gather(c, b):
        isl = pl.ds(c * CHUNK, CHUNK)
        return pltpu.async_copy(word_hbm.at[idx_w.at[isl]], rows[b], gsem[b])

    def wait_gather(c, b):
        isl = pl.ds(c * CHUNK, CHUNK)
        pltpu.make_async_copy(
            word_hbm.at[idx_w.at[isl]], rows[b], gsem[b]).wait()

    def out_slice(c):
        return out_hbm.at[pl.ds((wid * NCH + c) * CHUNK, CHUNK)]

    def compute_chunk(c, b):
        def group_body(g, carry):
            t0 = c * CHUNK + g * LANES
            pv = idx_p[pl.ds(t0, LANES)]
            sv = idx_s[pl.ds(t0, LANES)]
            ps_base = (pv * TYPE_VOCAB + sv) * H
            rowv = g * LANES + lane_iota
            ssum = [jnp.zeros((LANES,), jnp.float32) for _ in range(4)]
            ssq = [jnp.zeros((LANES,), jnp.float32) for _ in range(4)]
            for h in range(H):
                hv = jnp.full((LANES,), h, jnp.int32)
                w = plsc.load_gather(rows[b], [rowv, hv])
                p = plsc.load_gather(ps_v, [ps_base + h])
                acc = w + p
                tr_v[pl.ds(h * LANES, LANES)] = acc
                ssum[h % 4] = ssum[h % 4] + acc
                ssq[h % 4] = ssq[h % 4] + acc * acc
            mean = ((ssum[0] + ssum[1]) + (ssum[2] + ssum[3])) * (1.0 / H)
            ms = ((ssq[0] + ssq[1]) + (ssq[2] + ssq[3])) * (1.0 / H)
            inv = _rsqrt(ms - mean * mean + EPS)
            minv = mean * inv
            for tt in range(LANES):
                u = inv[tt]
                v = minv[tt]
                for j in range(HREG):
                    col = plsc.load_gather(
                        tr_v, [(j * LANES + lane_iota) * LANES + tt])
                    o = (col * u - v) * g_regs[j] + b_regs[j]
                    obuf[b][g * LANES + tt, pl.ds(j * LANES, LANES)] = o
            return carry

        lax.fori_loop(0, GROUPS, group_body, 0)

    issue_gather(0, 0)

    def pair_body(i, carry):
        for bb in range(2):
            c = i * 2 + bb
            wait_gather(c, bb)
            if bb == 0:
                issue_gather(c + 1, 1)
            else:
                @pl.when(i < NCH // 2 - 1)
                def _():
                    issue_gather(c + 1, 0)

            @pl.when(i >= 1)
            def _():
                pltpu.make_async_copy(obuf[bb], out_slice(c - 2),
                                      osem[bb]).wait()

            compute_chunk(c, bb)
            pltpu.async_copy(obuf[bb], out_slice(c), osem[bb])
        return carry

    lax.fori_loop(0, NCH // 2, pair_body, 0)
    pltpu.make_async_copy(obuf[0], out_slice(NCH - 2), osem[0]).wait()
    pltpu.make_async_copy(obuf[1], out_slice(NCH - 1), osem[1]).wait()


def kernel(x, pos_ids, sent_ids, word_W, pos_W, sent_W, gamma, beta):
    x2 = x.reshape(N).astype(jnp.int32)
    p2 = pos_ids.reshape(N).astype(jnp.int32)
    s2 = sent_ids.reshape(N).astype(jnp.int32)
    out = _sc_embed(x2, p2, s2, word_W, pos_W, sent_W, gamma, beta)
    return out.reshape(B, L, H)


# speedup vs baseline: 1.8945x; 1.5688x over previous
import functools

import jax
import jax.numpy as jnp
from jax import lax
from jax.experimental import pallas as pl
from jax.experimental.pallas import tpu as pltpu
from jax.experimental.pallas import tpu_sc as plsc

B, L, H = 1024, 200, 64
N = B * L
EPS = 1e-05

NC, NS, LANES = 2, 16, 16
NW = NC * NS
CHUNK = 128
GROUPS = CHUNK // LANES
PER_W = N // NW
NCH = PER_W // CHUNK
HREG = H // LANES
MAXLEN, TYPE_VOCAB = 200, 2
NPS = MAXLEN * TYPE_VOCAB


def _rsqrt(v):
    i = lax.bitcast_convert_type(v, jnp.int32)
    i = jnp.int32(0x5F3759DF) - lax.shift_right_logical(i, 1)
    y = lax.bitcast_convert_type(i, jnp.float32)
    half = v * 0.5
    for _ in range(3):
        y = y * (1.5 - half * y * y)
    return y


_mesh = plsc.VectorSubcoreMesh(core_axis_name="c", subcore_axis_name="s")


@functools.partial(
    pl.kernel,
    mesh=_mesh,
    out_type=jax.ShapeDtypeStruct((N, H), jnp.float32),
    compiler_params=pltpu.CompilerParams(
        needs_layout_passes=False, use_tc_tiling_on_sc=False),
    scratch_types=[
        pltpu.VMEM((PER_W,), jnp.int32),
        pltpu.VMEM((PER_W,), jnp.int32),
        pltpu.VMEM((PER_W,), jnp.int32),
        pltpu.VMEM((CHUNK, H), jnp.float32),
        pltpu.VMEM((CHUNK, H), jnp.float32),
        pltpu.VMEM((CHUNK, H), jnp.float32),
        pltpu.VMEM((CHUNK, H), jnp.float32),
        pltpu.VMEM((MAXLEN, H), jnp.float32),
        pltpu.VMEM((TYPE_VOCAB, H), jnp.float32),
        pltpu.VMEM((NPS * H,), jnp.float32),
        pltpu.VMEM((H,), jnp.float32),
        pltpu.VMEM((H,), jnp.float32),
        pltpu.SemaphoreType.DMA,
        pltpu.SemaphoreType.DMA,
        pltpu.SemaphoreType.DMA,
        pltpu.SemaphoreType.DMA,
    ],
)
def _sc_embed(x_hbm, pos_hbm, sent_hbm, word_hbm, posw_hbm, sentw_hbm,
              gamma_hbm, beta_hbm, out_hbm,
              idx_w, idx_p, idx_s, rows0, rows1, obuf0, obuf1,
              posw_v, sentw_v, ps_v, g_v, b_v,
              gsem0, gsem1, osem0, osem1):
    wid = lax.axis_index("s") * NC + lax.axis_index("c")
    rows = (rows0, rows1)
    obuf = (obuf0, obuf1)
    gsem = (gsem0, gsem1)
    osem = (osem0, osem1)

    pltpu.sync_copy(x_hbm.at[pl.ds(wid * PER_W, PER_W)], idx_w)
    pltpu.sync_copy(pos_hbm.at[pl.ds(wid * PER_W, PER_W)], idx_p)
    pltpu.sync_copy(sent_hbm.at[pl.ds(wid * PER_W, PER_W)], idx_s)
    pltpu.sync_copy(posw_hbm, posw_v)
    pltpu.sync_copy(sentw_hbm, sentw_v)
    pltpu.sync_copy(gamma_hbm, g_v)
    pltpu.sync_copy(beta_hbm, b_v)

    g_regs = [g_v[pl.ds(j * LANES, LANES)] for j in range(HREG)]
    b_regs = [b_v[pl.ds(j * LANES, LANES)] for j in range(HREG)]

    def ps_body(p, carry):
        for s in range(TYPE_VOCAB):
            base = (p * TYPE_VOCAB + s) * H
            for j in range(HREG):
                sl = pl.ds(j * LANES, LANES)
                ps_v[pl.ds(base + j * LANES, LANES)] = \
                    posw_v[p, sl] + sentw_v[s, sl]
        return carry

    lax.fori_loop(0, MAXLEN, ps_body, 0)

    def issue_gather(c, b):
        isl = pl.ds(c * CHUNK, CHUNK)
        return pltpu.async_copy(word_hbm.at[idx_w.at[isl]], rows[b], gsem[b])

    def wait_gather(c, b):
        isl = pl.ds(c * CHUNK, CHUNK)
        pltpu.make_async_copy(
            word_hbm.at[idx_w.at[isl]], rows[b], gsem[b]).wait()

    def out_slice(c):
        return out_hbm.at[pl.ds((wid * NCH + c) * CHUNK, CHUNK)]

    def compute_chunk(c, b):
        def group_body(g, carry):
            t0 = c * CHUNK + g * LANES
            pv = idx_p[pl.ds(t0, LANES)]
            sv = idx_s[pl.ds(t0, LANES)]
            ps_base = (pv * TYPE_VOCAB + sv) * H
            for tt in range(LANES):
                t = g * LANES + tt
                base = ps_base[tt]
                acc = []
                for j in range(HREG):
                    w = rows[b][t, pl.ds(j * LANES, LANES)]
                    p = ps_v[pl.ds(base + j * LANES, LANES)]
                    acc.append(w + p)
                tot = (acc[0] + acc[1]) + (acc[2] + acc[3])
                sq = (acc[0] * acc[0] + acc[1] * acc[1]) + \
                     (acc[2] * acc[2] + acc[3] * acc[3])
                s1 = lax.broadcast_in_dim(jnp.sum(tot), (LANES,), ())
                s2 = lax.broadcast_in_dim(jnp.sum(sq), (LANES,), ())
                mean = s1 * (1.0 / H)
                ms = s2 * (1.0 / H)
                inv = _rsqrt(ms - mean * mean + EPS)
                minv = mean * inv
                for j in range(HREG):
                    o = (acc[j] * inv - minv) * g_regs[j] + b_regs[j]
                    obuf[b][t, pl.ds(j * LANES, LANES)] = o
            return carry

        lax.fori_loop(0, GROUPS, group_body, 0)

    issue_gather(0, 0)

    def pair_body(i, carry):
        for bb in range(2):
            c = i * 2 + bb
            wait_gather(c, bb)
            if bb == 0:
                issue_gather(c + 1, 1)
            else:
                @pl.when(i < NCH // 2 - 1)
                def _():
                    issue_gather(c + 1, 0)

            @pl.when(i >= 1)
            def _():
                pltpu.make_async_copy(obuf[bb], out_slice(c - 2),
                                      osem[bb]).wait()

            compute_chunk(c, bb)
            pltpu.async_copy(obuf[bb], out_slice(c), osem[bb])
        return carry

    lax.fori_loop(0, NCH // 2, pair_body, 0)
    pltpu.make_async_copy(obuf[0], out_slice(NCH - 2), osem[0]).wait()
    pltpu.make_async_copy(obuf[1], out_slice(NCH - 1), osem[1]).wait()


def kernel(x, pos_ids, sent_ids, word_W, pos_W, sent_W, gamma, beta):
    x2 = x.reshape(N).astype(jnp.int32)
    p2 = pos_ids.reshape(N).astype(jnp.int32)
    s2 = sent_ids.reshape(N).astype(jnp.int32)
    out = _sc_embed(x2, p2, s2, word_W, pos_W, sent_W, gamma, beta)
    return out.reshape(B, L, H)


# speedup vs baseline: 1.8952x; 1.0004x over previous
import functools

import jax
import jax.numpy as jnp
from jax import lax
from jax.experimental import pallas as pl
from jax.experimental.pallas import tpu as pltpu
from jax.experimental.pallas import tpu_sc as plsc

B, L, H = 1024, 200, 64
N = B * L
EPS = 1e-05

NC, NS, LANES = 2, 16, 16
NW = NC * NS
LPAD = 256
ROWS_W = B // NW
GROUPS = L // LANES
TAIL = L - GROUPS * LANES
SPLIT = 96
HREG = H // LANES
MAXLEN, TYPE_VOCAB = 200, 2
NPS = MAXLEN * TYPE_VOCAB


def _rsqrt(v):
    i = lax.bitcast_convert_type(v, jnp.int32)
    i = jnp.int32(0x5F3759DF) - lax.shift_right_logical(i, 1)
    y = lax.bitcast_convert_type(i, jnp.float32)
    half = v * 0.5
    for _ in range(3):
        y = y * (1.5 - half * y * y)
    return y


_mesh = plsc.VectorSubcoreMesh(core_axis_name="c", subcore_axis_name="s")


@functools.partial(
    pl.kernel,
    mesh=_mesh,
    out_type=jax.ShapeDtypeStruct((N * H,), jnp.float32),
    compiler_params=pltpu.CompilerParams(
        needs_layout_passes=False, use_tc_tiling_on_sc=False),
    scratch_types=[
        pltpu.VMEM((ROWS_W * LPAD,), jnp.int32),
        pltpu.VMEM((ROWS_W * LPAD,), jnp.int32),
        pltpu.VMEM((ROWS_W * LPAD,), jnp.int32),
        pltpu.VMEM((L, H), jnp.float32),
        pltpu.VMEM((L, H), jnp.float32),
        pltpu.VMEM((L * H,), jnp.float32),
        pltpu.VMEM((L * H,), jnp.float32),
        pltpu.VMEM((TYPE_VOCAB, H), jnp.float32),
        pltpu.VMEM((NPS * H,), jnp.float32),
        pltpu.VMEM((H,), jnp.float32),
        pltpu.VMEM((H,), jnp.float32),
        pltpu.SemaphoreType.DMA,
        pltpu.SemaphoreType.DMA,
        pltpu.SemaphoreType.DMA,
        pltpu.SemaphoreType.DMA,
    ],
)
def _sc_embed(x_hbm, pos_hbm, sent_hbm, word_hbm, posw_hbm, sentw_hbm,
              gamma_hbm, beta_hbm, out_hbm,
              idx_w, idx_p, idx_s, rows0, rows1, obuf0, obuf1,
              sentw_v, ps_v, g_v, b_v,
              gsem0, gsem1, osem0, osem1):
    wid = lax.axis_index("s") * NC + lax.axis_index("c")
    rows = (rows0, rows1)
    obuf = (obuf0, obuf1)
    gsem = (gsem0, gsem1)
    osem = (osem0, osem1)

    pltpu.sync_copy(x_hbm.at[pl.ds(wid * ROWS_W * LPAD, ROWS_W * LPAD)], idx_w)
    pltpu.sync_copy(pos_hbm.at[pl.ds(wid * ROWS_W * LPAD, ROWS_W * LPAD)],
                    idx_p)
    pltpu.sync_copy(sent_hbm.at[pl.ds(wid * ROWS_W * LPAD, ROWS_W * LPAD)],
                    idx_s)
    pltpu.sync_copy(posw_hbm, rows0)
    pltpu.sync_copy(sentw_hbm, sentw_v)
    pltpu.sync_copy(gamma_hbm, g_v)
    pltpu.sync_copy(beta_hbm, b_v)

    g_regs = [g_v[pl.ds(j * LANES, LANES)] for j in range(HREG)]
    b_regs = [b_v[pl.ds(j * LANES, LANES)] for j in range(HREG)]

    def ps_body(p, carry):
        for s in range(TYPE_VOCAB):
            base = (p * TYPE_VOCAB + s) * H
            for j in range(HREG):
                sl = pl.ds(j * LANES, LANES)
                ps_v[pl.ds(base + j * LANES, LANES)] = \
                    rows0[p, sl] + sentw_v[s, sl]
        return carry

    lax.fori_loop(0, MAXLEN, ps_body, 0)

    def issue_gather(r, b):
        pltpu.async_copy(
            word_hbm.at[idx_w.at[pl.ds(r * LPAD, SPLIT)]],
            rows[b].at[pl.ds(0, SPLIT)], gsem[b])
        pltpu.async_copy(
            word_hbm.at[idx_w.at[pl.ds(r * LPAD + SPLIT, L - SPLIT)]],
            rows[b].at[pl.ds(SPLIT, L - SPLIT)], gsem[b])

    def wait_gather(r, b):
        pltpu.make_async_copy(
            word_hbm.at[idx_w.at[pl.ds(r * LPAD, SPLIT)]],
            rows[b].at[pl.ds(0, SPLIT)], gsem[b]).wait()
        pltpu.make_async_copy(
            word_hbm.at[idx_w.at[pl.ds(r * LPAD + SPLIT, L - SPLIT)]],
            rows[b].at[pl.ds(SPLIT, L - SPLIT)], gsem[b]).wait()

    def out_slice(r):
        return out_hbm.at[pl.ds((wid * ROWS_W + r) * L * H, L * H)]

    def compute_row(r, b):
        def group_body(g, n_tok, carry):
            pv = idx_p[pl.ds(r * LPAD + g * LANES, LANES)]
            sv = idx_s[pl.ds(r * LPAD + g * LANES, LANES)]
            ps_base = (pv * TYPE_VOCAB + sv) * H
            for tt in range(n_tok):
                t = g * LANES + tt
                base = ps_base[tt]
                acc = []
                for j in range(HREG):
                    w = rows[b][t, pl.ds(j * LANES, LANES)]
                    p = ps_v[pl.ds(base + j * LANES, LANES)]
                    acc.append(w + p)
                tot = (acc[0] + acc[1]) + (acc[2] + acc[3])
                sq = (acc[0] * acc[0] + acc[1] * acc[1]) + \
                     (acc[2] * acc[2] + acc[3] * acc[3])
                s1 = lax.broadcast_in_dim(jnp.sum(tot), (LANES,), ())
                s2 = lax.broadcast_in_dim(jnp.sum(sq), (LANES,), ())
                mean = s1 * (1.0 / H)
                ms = s2 * (1.0 / H)
                inv = _rsqrt(ms - mean * mean + EPS)
                minv = mean * inv
                for j in range(HREG):
                    o = (acc[j] * inv - minv) * g_regs[j] + b_regs[j]
                    obuf[b][pl.ds(t * H + j * LANES, LANES)] = o
            return carry

        lax.fori_loop(0, GROUPS,
                      lambda g, cy: group_body(g, LANES, cy), 0)
        group_body(GROUPS, TAIL, 0)

    issue_gather(0, 0)

    def pair_body(i, carry):
        for bb in range(2):
            r = i * 2 + bb
            wait_gather(r, bb)
            if bb == 0:
                issue_gather(r + 1, 1)
            else:
                @pl.when(i < ROWS_W // 2 - 1)
                def _():
                    issue_gather(r + 1, 0)

            @pl.when(i >= 1)
            def _():
                pltpu.make_async_copy(obuf[bb], out_slice(r - 2),
                                      osem[bb]).wait()

            compute_row(r, bb)
            pltpu.async_copy(obuf[bb], out_slice(r), osem[bb])
        return carry

    lax.fori_loop(0, ROWS_W // 2, pair_body, 0)
    pltpu.make_async_copy(obuf[0], out_slice(ROWS_W - 2), osem[0]).wait()
    pltpu.make_async_copy(obuf[1], out_slice(ROWS_W - 1), osem[1]).wait()


def kernel(x, pos_ids, sent_ids, word_W, pos_W, sent_W, gamma, beta):
    pad = ((0, 0), (0, LPAD - L))
    xp = jnp.pad(x.astype(jnp.int32), pad).reshape(B * LPAD)
    pp = jnp.pad(pos_ids.astype(jnp.int32), pad).reshape(B * LPAD)
    sp = jnp.pad(sent_ids.astype(jnp.int32), pad).reshape(B * LPAD)
    out = _sc_embed(xp, pp, sp, word_W, pos_W, sent_W, gamma, beta)
    return out.reshape(B, L, H)
